# baseline (speedup 1.0000x reference)
import functools

import jax
import jax.numpy as jnp
from jax import lax
from jax.experimental import pallas as pl
from jax.experimental.pallas import tpu as pltpu
from jax.experimental.pallas import tpu_sc as plsc

N = 10000
K = 16
L = 32
D = 128
NLAYER = 3
NG = 64

SC_NC = 2
SC_NS = 16
NW = SC_NC * SC_NS
NPADSC = 10240
VPW = 5120
BPAD = NW * VPW
NV16 = VPW // 16

BN = 400
NB = N // BN
BSUB = 8
NSUB = BN // BSUB
SPB = BSUB * K
EPB = BSUB * L
BNK = BN * K
BNL = BN * L


def _sc_gather_body(x_hbm, idx_hbm, out_hbm, x_v, idx_v, out_v):
    wid = lax.axis_index("s") * SC_NC + lax.axis_index("c")
    pltpu.sync_copy(x_hbm, x_v)
    pltpu.sync_copy(idx_hbm.at[pl.ds(wid * VPW, VPW)], idx_v)

    def body(m, carry):
        iv = idx_v[pl.ds(m * 16, 16)]
        out_v[pl.ds(m * 16, 16)] = plsc.load_gather(
            x_v, [lax.shift_right_logical(iv, 7), lax.bitwise_and(iv, 127)])
        return carry

    lax.fori_loop(0, NV16, body, 0, unroll=8)
    pltpu.sync_copy(out_v, out_hbm.at[pl.ds(wid * VPW, VPW)])


@functools.cache
def _sc_gather():
    return pl.kernel(
        _sc_gather_body,
        out_type=jax.ShapeDtypeStruct((BPAD,), jnp.int32),
        mesh=plsc.VectorSubcoreMesh(core_axis_name="c", subcore_axis_name="s"),
        scratch_types=[
            pltpu.VMEM((NPADSC // 128, 128), jnp.int32),
            pltpu.VMEM((VPW,), jnp.int32),
            pltpu.VMEM((VPW,), jnp.int32),
        ],
        compiler_params=pltpu.CompilerParams(needs_layout_passes=False),
    )


def _main_body(xrep_ref, gx_ref, xval_ref, gsrc_ref, gdst_ref, attr_ref, batch_ref,
               xtab_ref, eatab_ref, tftab_ref, l0w_ref, l0b_ref, l1w_ref, l1b_ref,
               cw1_ref, cb1_ref, cw2_ref, cb2_ref, pw_ref, pb_ref,
               out_ref, x_sc, ea_sc, agg_sc, hg_sc):
    f32 = jnp.float32
    b = pl.program_id(0)

    t0tab = jnp.dot(xtab_ref[...], l0w_ref[...], preferred_element_type=f32) + l0b_ref[...]
    t1tab = jnp.dot(xtab_ref[...], l1w_ref[...], preferred_element_type=f32) + l1b_ref[...]

    oha = (attr_ref[...] == lax.broadcasted_iota(jnp.int32, (BNL, 16), 1)).astype(f32)
    ea_sc[...] = jnp.dot(oha, eatab_ref[...], preferred_element_type=f32)

    oh0 = (xrep_ref[...] == lax.broadcasted_iota(jnp.int32, (BNK, 32), 1)).astype(f32)
    oh1 = (gx_ref[...] == lax.broadcasted_iota(jnp.int32, (BNK, 32), 1)).astype(f32)
    oht = (xval_ref[...] == lax.broadcasted_iota(jnp.int32, (BNK, 16), 1)).astype(f32)
    t0b = jnp.dot(oh0, t0tab, preferred_element_type=f32)
    t1b = jnp.dot(oh1, t1tab, preferred_element_type=f32)
    xt = jnp.dot(oht, tftab_ref[...], preferred_element_type=f32)
    x_sc[...] = t0b * t1b * xt

    for l in range(NLAYER):
        def sub(s, carry):
            gs = gsrc_ref[pl.ds(s * EPB, EPB), :]
            ohs = (gs == lax.broadcasted_iota(jnp.int32, (EPB, SPB), 1)).astype(f32)
            xs = x_sc[pl.ds(s * SPB, SPB), :]
            srcf = jnp.dot(ohs, xs, preferred_element_type=f32)
            msg = srcf * ea_sc[pl.ds(s * EPB, EPB), :]
            gd = gdst_ref[:, pl.ds(s, 1), :].reshape(1, EPB)
            ohd = (lax.broadcasted_iota(jnp.int32, (SPB, EPB), 0) == gd).astype(f32)
            agg_sc[pl.ds(s * SPB, SPB), :] = jnp.dot(ohd, msg, preferred_element_type=f32)
            return carry

        lax.fori_loop(0, NSUB, sub, 0, unroll=2)

        h = jnp.maximum(
            jnp.dot(agg_sc[...], cw1_ref[l], preferred_element_type=f32)
            + cb1_ref[l:l + 1, :], 0.0)
        x_sc[...] = (x_sc[...]
                     + jnp.dot(h, cw2_ref[l], preferred_element_type=f32)
                     + cb2_ref[l:l + 1, :])

    x3 = x_sc[...].reshape(BN, K, D)
    xnode = x3[:, 0, :]
    for k in range(1, K):
        xnode = jnp.maximum(xnode, x3[:, k, :])

    bt = batch_ref[...].reshape(1, BN)
    ohb = (lax.broadcasted_iota(jnp.int32, (NG, BN), 0) == bt).astype(f32)
    contrib = jnp.dot(ohb, xnode, preferred_element_type=f32)

    @pl.when(b == 0)
    def _():
        hg_sc[...] = contrib

    @pl.when(b > 0)
    def _():
        hg_sc[...] = hg_sc[...] + contrib

    @pl.when(b == NB - 1)
    def _():
        out_ref[...] = (jnp.dot(hg_sc[...], pw_ref[...], preferred_element_type=f32)
                        + pb_ref[...])


def _full(shape):
    return pl.BlockSpec(shape, lambda i: (0,) * len(shape))


_main_call = pl.pallas_call(
    _main_body,
    grid=(NB,),
    in_specs=[
        pl.BlockSpec((BNK, 1), lambda i: (i, 0)),
        pl.BlockSpec((BNK, 1), lambda i: (i, 0)),
        pl.BlockSpec((BNK, 1), lambda i: (i, 0)),
        pl.BlockSpec((BNL, 1), lambda i: (i, 0)),
        pl.BlockSpec((1, NSUB, EPB), lambda i: (i, 0, 0)),
        pl.BlockSpec((BNL, 1), lambda i: (i, 0)),
        pl.BlockSpec((1, 1, BN), lambda i: (i, 0, 0)),
        _full((32, D)), _full((16, D)), _full((16, D)),
        _full((D, D)), _full((1, D)), _full((D, D)), _full((1, D)),
        _full((NLAYER, D, D)), _full((NLAYER, D)),
        _full((NLAYER, D, D)), _full((NLAYER, D)),
        _full((D, 1)), _full((1, 1)),
    ],
    out_specs=pl.BlockSpec((NG, 1), lambda i: (0, 0)),
    out_shape=jax.ShapeDtypeStruct((NG, 1), jnp.float32),
    scratch_shapes=[
        pltpu.VMEM((BNK, D), jnp.float32),
        pltpu.VMEM((BNL, D), jnp.float32),
        pltpu.VMEM((BNK, D), jnp.float32),
        pltpu.VMEM((NG, D), jnp.float32),
    ],
    compiler_params=pltpu.CompilerParams(
        dimension_semantics=("arbitrary",),
        vmem_limit_bytes=100 * 1024 * 1024,
    ),
)


def kernel(x, subg_nodes, local_src, local_dst, local_attr, X_val, batch,
           x_table, ea_table, tf_table, lin0_W, lin0_b, lin1_W, lin1_b,
           conv_W1, conv_b1, conv_W2, conv_b2, pred_W, pred_b):
    i32 = jnp.int32
    xi = x.astype(i32)
    xi_pad = jnp.concatenate([xi, jnp.zeros((NPADSC - N,), i32)]).reshape(NPADSC // 128, 128)
    idx = subg_nodes.astype(i32).reshape(N * K)
    idx = jnp.concatenate([idx, jnp.zeros((BPAD - N * K,), i32)])
    gx = _sc_gather()(xi_pad, idx)[:N * K].reshape(N * K, 1)

    xrep = jnp.repeat(xi, K).reshape(N * K, 1)
    roff = (jnp.arange(N, dtype=i32)[:, None] % BSUB) * K
    gsrc = (roff + local_src.astype(i32)).reshape(N * L, 1)
    gdst = (roff + local_dst.astype(i32)).reshape(NB, NSUB, EPB)
    out = _main_call(
        xrep, gx,
        X_val.astype(i32).reshape(N * K, 1),
        gsrc, gdst,
        local_attr.astype(i32).reshape(N * L, 1),
        batch.astype(i32).reshape(NB, 1, BN),
        x_table, ea_table, tf_table,
        lin0_W, lin0_b.reshape(1, D), lin1_W, lin1_b.reshape(1, D),
        conv_W1, conv_b1, conv_W2, conv_b2,
        pred_W, pred_b.reshape(1, 1),
    )
    return out


# speedup vs baseline: 1.4947x; 1.4947x over previous
import functools

import jax
import jax.numpy as jnp
from jax import lax
from jax.experimental import pallas as pl
from jax.experimental.pallas import tpu as pltpu
from jax.experimental.pallas import tpu_sc as plsc

N = 10000
K = 16
L = 32
D = 128
NLAYER = 3
NG = 64

SC_NC = 2
SC_NS = 16
NW = SC_NC * SC_NS
NPADSC = 10240
VPW = 5120
BPAD = NW * VPW
NV16 = VPW // 16

BN = 400
NB = N // BN
BSUB = 8
NSUB = BN // BSUB
SPB = BSUB * K
EPB = BSUB * L
BNK = BN * K
BNL = BN * L

_TDIMS = (((0,), (0,)), ((), ()))


def _sc_gather_body(x_hbm, idx_hbm, out_hbm, x_v, idx_v, out_v):
    wid = lax.axis_index("s") * SC_NC + lax.axis_index("c")
    pltpu.sync_copy(x_hbm, x_v)
    pltpu.sync_copy(idx_hbm.at[pl.ds(wid * VPW, VPW)], idx_v)

    def body(m, carry):
        iv = idx_v[pl.ds(m * 16, 16)]
        out_v[pl.ds(m * 16, 16)] = plsc.load_gather(
            x_v, [lax.shift_right_logical(iv, 7), lax.bitwise_and(iv, 127)])
        return carry

    lax.fori_loop(0, NV16, body, 0, unroll=8)
    pltpu.sync_copy(out_v, out_hbm.at[pl.ds(wid * VPW, VPW)])


@functools.cache
def _sc_gather():
    return pl.kernel(
        _sc_gather_body,
        out_type=jax.ShapeDtypeStruct((BPAD,), jnp.int32),
        mesh=plsc.VectorSubcoreMesh(core_axis_name="c", subcore_axis_name="s"),
        scratch_types=[
            pltpu.VMEM((NPADSC // 128, 128), jnp.int32),
            pltpu.VMEM((VPW,), jnp.int32),
            pltpu.VMEM((VPW,), jnp.int32),
        ],
        compiler_params=pltpu.CompilerParams(needs_layout_passes=False),
    )


def _main_body(xrep_ref, gx_ref, xval_ref, gsrc_ref, gdst_ref, attr_ref, batch_ref,
               xtab_ref, eatab_ref, tftab_ref, l0w_ref, l0b_ref, l1w_ref, l1b_ref,
               cw1_ref, cb1_ref, cw2_ref, cb2_ref, pw_ref, pb_ref,
               out_ref, x_sc, ea_sc, agg_sc, hg_sc):
    f32 = jnp.float32
    i32 = jnp.int32
    b = pl.program_id(0)

    t0tab = jnp.dot(xtab_ref[...], l0w_ref[...], preferred_element_type=f32) + l0b_ref[...]
    t1tab = jnp.dot(xtab_ref[...], l1w_ref[...], preferred_element_type=f32) + l1b_ref[...]

    oha = (lax.broadcasted_iota(i32, (16, BNL), 0)
           == attr_ref[...].reshape(1, BNL)).astype(f32)
    ea_sc[...] = lax.dot_general(oha, eatab_ref[...], _TDIMS,
                                 preferred_element_type=f32)

    oh0 = (lax.broadcasted_iota(i32, (32, BNK), 0)
           == xrep_ref[...].reshape(1, BNK)).astype(f32)
    oh1 = (lax.broadcasted_iota(i32, (32, BNK), 0)
           == gx_ref[...].reshape(1, BNK)).astype(f32)
    oht = (lax.broadcasted_iota(i32, (16, BNK), 0)
           == xval_ref[...].reshape(1, BNK)).astype(f32)
    t0b = lax.dot_general(oh0, t0tab, _TDIMS, preferred_element_type=f32)
    t1b = lax.dot_general(oh1, t1tab, _TDIMS, preferred_element_type=f32)
    xt = lax.dot_general(oht, tftab_ref[...], _TDIMS, preferred_element_type=f32)
    x_sc[...] = t0b * t1b * xt

    for l in range(NLAYER):
        def sub(s, carry):
            gs = gsrc_ref[:, :, pl.ds(s * EPB, EPB)].reshape(1, EPB)
            ohs = (lax.broadcasted_iota(i32, (SPB, EPB), 0) == gs).astype(f32)
            xs = x_sc[pl.ds(s * SPB, SPB), :]
            srcf = lax.dot_general(ohs, xs, _TDIMS, preferred_element_type=f32)
            msg = srcf * ea_sc[pl.ds(s * EPB, EPB), :]
            gd = gdst_ref[:, :, pl.ds(s * EPB, EPB)].reshape(1, EPB)
            ohd = (lax.broadcasted_iota(i32, (SPB, EPB), 0) == gd).astype(f32)
            agg_sc[pl.ds(s * SPB, SPB), :] = jnp.dot(ohd, msg, preferred_element_type=f32)
            return carry

        lax.fori_loop(0, NSUB, sub, 0, unroll=2)

        h = jnp.maximum(
            jnp.dot(agg_sc[...], cw1_ref[l], preferred_element_type=f32)
            + cb1_ref[l:l + 1, :], 0.0)
        x_sc[...] = (x_sc[...]
                     + jnp.dot(h, cw2_ref[l], preferred_element_type=f32)
                     + cb2_ref[l:l + 1, :])

    x3 = x_sc[...].reshape(BN, K, D)
    xnode = x3[:, 0, :]
    for k in range(1, K):
        xnode = jnp.maximum(xnode, x3[:, k, :])

    bt = batch_ref[...].reshape(1, BN)
    ohb = (lax.broadcasted_iota(i32, (NG, BN), 0) == bt).astype(f32)
    contrib = jnp.dot(ohb, xnode, preferred_element_type=f32)

    @pl.when(b == 0)
    def _():
        hg_sc[...] = contrib

    @pl.when(b > 0)
    def _():
        hg_sc[...] = hg_sc[...] + contrib

    @pl.when(b == NB - 1)
    def _():
        out_ref[...] = (jnp.dot(hg_sc[...], pw_ref[...], preferred_element_type=f32)
                        + pb_ref[...])


def _full(shape):
    return pl.BlockSpec(shape, lambda i: (0,) * len(shape))


_main_call = pl.pallas_call(
    _main_body,
    grid=(NB,),
    in_specs=[
        pl.BlockSpec((1, 1, BNK), lambda i: (i, 0, 0)),
        pl.BlockSpec((1, 1, BNK), lambda i: (i, 0, 0)),
        pl.BlockSpec((1, 1, BNK), lambda i: (i, 0, 0)),
        pl.BlockSpec((1, 1, BNL), lambda i: (i, 0, 0)),
        pl.BlockSpec((1, 1, BNL), lambda i: (i, 0, 0)),
        pl.BlockSpec((1, 1, BNL), lambda i: (i, 0, 0)),
        pl.BlockSpec((1, 1, BN), lambda i: (i, 0, 0)),
        _full((32, D)), _full((16, D)), _full((16, D)),
        _full((D, D)), _full((1, D)), _full((D, D)), _full((1, D)),
        _full((NLAYER, D, D)), _full((NLAYER, D)),
        _full((NLAYER, D, D)), _full((NLAYER, D)),
        _full((D, 1)), _full((1, 1)),
    ],
    out_specs=pl.BlockSpec((NG, 1), lambda i: (0, 0)),
    out_shape=jax.ShapeDtypeStruct((NG, 1), jnp.float32),
    scratch_shapes=[
        pltpu.VMEM((BNK, D), jnp.float32),
        pltpu.VMEM((BNL, D), jnp.float32),
        pltpu.VMEM((BNK, D), jnp.float32),
        pltpu.VMEM((NG, D), jnp.float32),
    ],
    compiler_params=pltpu.CompilerParams(
        dimension_semantics=("arbitrary",),
        vmem_limit_bytes=100 * 1024 * 1024,
        fuse_transposed_lhs_in_matmul=True,
    ),
)


def kernel(x, subg_nodes, local_src, local_dst, local_attr, X_val, batch,
           x_table, ea_table, tf_table, lin0_W, lin0_b, lin1_W, lin1_b,
           conv_W1, conv_b1, conv_W2, conv_b2, pred_W, pred_b):
    i32 = jnp.int32
    xi = x.astype(i32)
    xi_pad = jnp.concatenate([xi, jnp.zeros((NPADSC - N,), i32)]).reshape(NPADSC // 128, 128)
    idx = subg_nodes.astype(i32).reshape(N * K)
    idx = jnp.concatenate([idx, jnp.zeros((BPAD - N * K,), i32)])
    gx = _sc_gather()(xi_pad, idx)[:N * K].reshape(NB, 1, BNK)

    xrep = jnp.repeat(xi, K).reshape(NB, 1, BNK)
    roff = (jnp.arange(N, dtype=i32)[:, None] % BSUB) * K
    gsrc = (roff + local_src.astype(i32)).reshape(NB, 1, BNL)
    gdst = (roff + local_dst.astype(i32)).reshape(NB, 1, BNL)
    out = _main_call(
        xrep, gx,
        X_val.astype(i32).reshape(NB, 1, BNK),
        gsrc, gdst,
        local_attr.astype(i32).reshape(NB, 1, BNL),
        batch.astype(i32).reshape(NB, 1, BN),
        x_table, ea_table, tf_table,
        lin0_W, lin0_b.reshape(1, D), lin1_W, lin1_b.reshape(1, D),
        conv_W1, conv_b1, conv_W2, conv_b2,
        pred_W, pred_b.reshape(1, 1),
    )
    return out


# speedup vs baseline: 1.5293x; 1.0231x over previous
import functools

import jax
import jax.numpy as jnp
from jax import lax
from jax.experimental import pallas as pl
from jax.experimental.pallas import tpu as pltpu
from jax.experimental.pallas import tpu_sc as plsc

N = 10000
K = 16
L = 32
D = 128
NLAYER = 3
NG = 64

SC_NC = 2
SC_NS = 16
NW = SC_NC * SC_NS
NPADSC = 10240
VPW = 5120
BPAD = NW * VPW
NV16 = VPW // 16

BN = 400
NB = N // BN
BSUB = 8
NSUB = BN // BSUB
SPB = BSUB * K
EPB = BSUB * L
BNK = BN * K
BNL = BN * L

_TDIMS = (((0,), (0,)), ((), ()))


def _sc_gather_body(x_hbm, idx_hbm, out_hbm, x_v, idx_v, out_v):
    wid = lax.axis_index("s") * SC_NC + lax.axis_index("c")
    pltpu.sync_copy(x_hbm, x_v)
    pltpu.sync_copy(idx_hbm.at[pl.ds(wid * VPW, VPW)], idx_v)

    def body(m, carry):
        iv = idx_v[pl.ds(m * 16, 16)]
        out_v[pl.ds(m * 16, 16)] = plsc.load_gather(
            x_v, [lax.shift_right_logical(iv, 7), lax.bitwise_and(iv, 127)])
        return carry

    lax.fori_loop(0, NV16, body, 0, unroll=8)
    pltpu.sync_copy(out_v, out_hbm.at[pl.ds(wid * VPW, VPW)])


@functools.cache
def _sc_gather():
    return pl.kernel(
        _sc_gather_body,
        out_type=jax.ShapeDtypeStruct((BPAD,), jnp.int32),
        mesh=plsc.VectorSubcoreMesh(core_axis_name="c", subcore_axis_name="s"),
        scratch_types=[
            pltpu.VMEM((NPADSC // 128, 128), jnp.int32),
            pltpu.VMEM((VPW,), jnp.int32),
            pltpu.VMEM((VPW,), jnp.int32),
        ],
        compiler_params=pltpu.CompilerParams(needs_layout_passes=False),
    )


def _main_body(xrep_ref, gx_ref, xval_ref, gsrc_ref, gdst_ref, attr_ref, batch_ref,
               xtab_ref, eatab_ref, tftab_ref, l0w_ref, l0b_ref, l1w_ref, l1b_ref,
               cw1_ref, cb1_ref, cw2_ref, cb2_ref, pw_ref, pb_ref,
               out_ref, x_sc, ea_sc, agg_sc, hg_sc, ohs_sc, ohd_sc):
    f32 = jnp.float32
    i32 = jnp.int32
    bf = jnp.bfloat16
    b = pl.program_id(0)

    t0tab = (jnp.dot(xtab_ref[...], l0w_ref[...], preferred_element_type=f32)
             + l0b_ref[...]).astype(bf)
    t1tab = (jnp.dot(xtab_ref[...], l1w_ref[...], preferred_element_type=f32)
             + l1b_ref[...]).astype(bf)

    oha = (lax.broadcasted_iota(i32, (16, BNL), 0)
           == attr_ref[...].reshape(1, BNL)).astype(bf)
    ea_sc[...] = lax.dot_general(oha, eatab_ref[...].astype(bf), _TDIMS,
                                 preferred_element_type=f32)

    oh0 = (lax.broadcasted_iota(i32, (32, BNK), 0)
           == xrep_ref[...].reshape(1, BNK)).astype(bf)
    oh1 = (lax.broadcasted_iota(i32, (32, BNK), 0)
           == gx_ref[...].reshape(1, BNK)).astype(bf)
    oht = (lax.broadcasted_iota(i32, (16, BNK), 0)
           == xval_ref[...].reshape(1, BNK)).astype(bf)
    t0b = lax.dot_general(oh0, t0tab, _TDIMS, preferred_element_type=f32)
    t1b = lax.dot_general(oh1, t1tab, _TDIMS, preferred_element_type=f32)
    xt = lax.dot_general(oht, tftab_ref[...].astype(bf), _TDIMS,
                         preferred_element_type=f32)
    x_sc[...] = t0b * t1b * xt

    def build(s, carry):
        gs = gsrc_ref[:, :, pl.ds(s * EPB, EPB)].reshape(1, EPB)
        ohs_sc[pl.ds(s * SPB, SPB), :] = (
            lax.broadcasted_iota(i32, (SPB, EPB), 0) == gs).astype(bf)
        gd = gdst_ref[:, :, pl.ds(s * EPB, EPB)].reshape(1, EPB)
        ohd_sc[pl.ds(s * SPB, SPB), :] = (
            lax.broadcasted_iota(i32, (SPB, EPB), 0) == gd).astype(bf)
        return carry

    lax.fori_loop(0, NSUB, build, 0, unroll=2)

    for l in range(NLAYER):
        def sub(s, carry):
            xs = x_sc[pl.ds(s * SPB, SPB), :].astype(bf)
            srcf = lax.dot_general(ohs_sc[pl.ds(s * SPB, SPB), :], xs, _TDIMS,
                                   preferred_element_type=f32)
            msg = (srcf * ea_sc[pl.ds(s * EPB, EPB), :]).astype(bf)
            agg_sc[pl.ds(s * SPB, SPB), :] = jnp.dot(
                ohd_sc[pl.ds(s * SPB, SPB), :], msg, preferred_element_type=f32)
            return carry

        lax.fori_loop(0, NSUB, sub, 0, unroll=2)

        h = jnp.maximum(
            jnp.dot(agg_sc[...].astype(bf), cw1_ref[l].astype(bf),
                    preferred_element_type=f32)
            + cb1_ref[l:l + 1, :], 0.0)
        x_sc[...] = (x_sc[...]
                     + jnp.dot(h.astype(bf), cw2_ref[l].astype(bf),
                               preferred_element_type=f32)
                     + cb2_ref[l:l + 1, :])

    x3 = x_sc[...].reshape(BN, K, D)
    xnode = x3[:, 0, :]
    for k in range(1, K):
        xnode = jnp.maximum(xnode, x3[:, k, :])

    bt = batch_ref[...].reshape(1, BN)
    ohb = (lax.broadcasted_iota(i32, (NG, BN), 0) == bt).astype(f32)
    contrib = jnp.dot(ohb, xnode, preferred_element_type=f32)

    @pl.when(b == 0)
    def _():
        hg_sc[...] = contrib

    @pl.when(b > 0)
    def _():
        hg_sc[...] = hg_sc[...] + contrib

    @pl.when(b == NB - 1)
    def _():
        out_ref[...] = (jnp.dot(hg_sc[...], pw_ref[...], preferred_element_type=f32)
                        + pb_ref[...])


def _full(shape):
    return pl.BlockSpec(shape, lambda i: (0,) * len(shape))


_main_call = pl.pallas_call(
    _main_body,
    grid=(NB,),
    in_specs=[
        pl.BlockSpec((1, 1, BNK), lambda i: (i, 0, 0)),
        pl.BlockSpec((1, 1, BNK), lambda i: (i, 0, 0)),
        pl.BlockSpec((1, 1, BNK), lambda i: (i, 0, 0)),
        pl.BlockSpec((1, 1, BNL), lambda i: (i, 0, 0)),
        pl.BlockSpec((1, 1, BNL), lambda i: (i, 0, 0)),
        pl.BlockSpec((1, 1, BNL), lambda i: (i, 0, 0)),
        pl.BlockSpec((1, 1, BN), lambda i: (i, 0, 0)),
        _full((32, D)), _full((16, D)), _full((16, D)),
        _full((D, D)), _full((1, D)), _full((D, D)), _full((1, D)),
        _full((NLAYER, D, D)), _full((NLAYER, D)),
        _full((NLAYER, D, D)), _full((NLAYER, D)),
        _full((D, 1)), _full((1, 1)),
    ],
    out_specs=pl.BlockSpec((NG, 1), lambda i: (0, 0)),
    out_shape=jax.ShapeDtypeStruct((NG, 1), jnp.float32),
    scratch_shapes=[
        pltpu.VMEM((BNK, D), jnp.float32),
        pltpu.VMEM((BNL, D), jnp.float32),
        pltpu.VMEM((BNK, D), jnp.float32),
        pltpu.VMEM((NG, D), jnp.float32),
        pltpu.VMEM((NSUB * SPB, EPB), jnp.bfloat16),
        pltpu.VMEM((NSUB * SPB, EPB), jnp.bfloat16),
    ],
    compiler_params=pltpu.CompilerParams(
        dimension_semantics=("arbitrary",),
        vmem_limit_bytes=100 * 1024 * 1024,
        fuse_transposed_lhs_in_matmul=True,
    ),
)


def kernel(x, subg_nodes, local_src, local_dst, local_attr, X_val, batch,
           x_table, ea_table, tf_table, lin0_W, lin0_b, lin1_W, lin1_b,
           conv_W1, conv_b1, conv_W2, conv_b2, pred_W, pred_b):
    i32 = jnp.int32
    xi = x.astype(i32)
    xi_pad = jnp.concatenate([xi, jnp.zeros((NPADSC - N,), i32)]).reshape(NPADSC // 128, 128)
    idx = subg_nodes.astype(i32).reshape(N * K)
    idx = jnp.concatenate([idx, jnp.zeros((BPAD - N * K,), i32)])
    gx = _sc_gather()(xi_pad, idx)[:N * K].reshape(NB, 1, BNK)

    xrep = jnp.repeat(xi, K).reshape(NB, 1, BNK)
    roff = (jnp.arange(N, dtype=i32)[:, None] % BSUB) * K
    gsrc = (roff + local_src.astype(i32)).reshape(NB, 1, BNL)
    gdst = (roff + local_dst.astype(i32)).reshape(NB, 1, BNL)
    out = _main_call(
        xrep, gx,
        X_val.astype(i32).reshape(NB, 1, BNK),
        gsrc, gdst,
        local_attr.astype(i32).reshape(NB, 1, BNL),
        batch.astype(i32).reshape(NB, 1, BN),
        x_table, ea_table, tf_table,
        lin0_W, lin0_b.reshape(1, D), lin1_W, lin1_b.reshape(1, D),
        conv_W1, conv_b1, conv_W2, conv_b2,
        pred_W, pred_b.reshape(1, 1),
    )
    return out


# speedup vs baseline: 1.5590x; 1.0194x over previous
import functools

import jax
import jax.numpy as jnp
from jax import lax
from jax.experimental import pallas as pl
from jax.experimental.pallas import tpu as pltpu
from jax.experimental.pallas import tpu_sc as plsc

N = 10000
K = 16
L = 32
D = 128
NLAYER = 3
NG = 64

SC_NC = 2
SC_NS = 16
NW = SC_NC * SC_NS
NPADSC = 10240
VPW = 5120
BPAD = NW * VPW
NV16 = VPW // 16

BN = 1000
NB = N // BN
BSUB = 8
NSUB = BN // BSUB
SPB = BSUB * K
EPB = BSUB * L
BNK = BN * K
BNL = BN * L

_TDIMS = (((0,), (0,)), ((), ()))


def _sc_gather_body(x_hbm, idx_hbm, out_hbm, x_v, idx_v, out_v):
    wid = lax.axis_index("s") * SC_NC + lax.axis_index("c")
    pltpu.sync_copy(x_hbm, x_v)
    pltpu.sync_copy(idx_hbm.at[pl.ds(wid * VPW, VPW)], idx_v)

    def body(m, carry):
        iv = idx_v[pl.ds(m * 16, 16)]
        out_v[pl.ds(m * 16, 16)] = plsc.load_gather(
            x_v, [lax.shift_right_logical(iv, 7), lax.bitwise_and(iv, 127)])
        return carry

    lax.fori_loop(0, NV16, body, 0, unroll=8)
    pltpu.sync_copy(out_v, out_hbm.at[pl.ds(wid * VPW, VPW)])


@functools.cache
def _sc_gather():
    return pl.kernel(
        _sc_gather_body,
        out_type=jax.ShapeDtypeStruct((BPAD,), jnp.int32),
        mesh=plsc.VectorSubcoreMesh(core_axis_name="c", subcore_axis_name="s"),
        scratch_types=[
            pltpu.VMEM((NPADSC // 128, 128), jnp.int32),
            pltpu.VMEM((VPW,), jnp.int32),
            pltpu.VMEM((VPW,), jnp.int32),
        ],
        compiler_params=pltpu.CompilerParams(needs_layout_passes=False),
    )


def _main_body(xrep_ref, gx_ref, xval_ref, gsrc_ref, gdst_ref, attr_ref, batch_ref,
               xtab_ref, eatab_ref, tftab_ref, l0w_ref, l0b_ref, l1w_ref, l1b_ref,
               cw1_ref, cb1_ref, cw2_ref, cb2_ref, pw_ref, pb_ref,
               out_ref, x_sc, xb_sc, ea_sc, agg_sc, hg_sc, ohs_sc, ohd_sc):
    f32 = jnp.float32
    i32 = jnp.int32
    bf = jnp.bfloat16
    b = pl.program_id(0)

    t0tab = (jnp.dot(xtab_ref[...], l0w_ref[...], preferred_element_type=f32)
             + l0b_ref[...]).astype(bf)
    t1tab = (jnp.dot(xtab_ref[...], l1w_ref[...], preferred_element_type=f32)
             + l1b_ref[...]).astype(bf)

    oha = (lax.broadcasted_iota(i32, (16, BNL), 0)
           == attr_ref[...].reshape(1, BNL)).astype(bf)
    ea_sc[...] = lax.dot_general(oha, eatab_ref[...].astype(bf), _TDIMS,
                                 preferred_element_type=f32).astype(bf)

    oh0 = (lax.broadcasted_iota(i32, (32, BNK), 0)
           == xrep_ref[...].reshape(1, BNK)).astype(bf)
    oh1 = (lax.broadcasted_iota(i32, (32, BNK), 0)
           == gx_ref[...].reshape(1, BNK)).astype(bf)
    oht = (lax.broadcasted_iota(i32, (16, BNK), 0)
           == xval_ref[...].reshape(1, BNK)).astype(bf)
    t0b = lax.dot_general(oh0, t0tab, _TDIMS, preferred_element_type=f32)
    t1b = lax.dot_general(oh1, t1tab, _TDIMS, preferred_element_type=f32)
    xt = lax.dot_general(oht, tftab_ref[...].astype(bf), _TDIMS,
                         preferred_element_type=f32)
    x_sc[...] = t0b * t1b * xt

    def build(s, carry):
        gs = gsrc_ref[:, :, pl.ds(s * EPB, EPB)].reshape(1, EPB)
        ohs_sc[pl.ds(s * SPB, SPB), :] = (
            lax.broadcasted_iota(i32, (SPB, EPB), 0) == gs).astype(bf)
        gd = gdst_ref[:, :, pl.ds(s * EPB, EPB)].reshape(1, EPB)
        ohd_sc[pl.ds(s * SPB, SPB), :] = (
            lax.broadcasted_iota(i32, (SPB, EPB), 0) == gd).astype(bf)
        return carry

    lax.fori_loop(0, NSUB, build, 0, unroll=2)

    for l in range(NLAYER):
        xb_sc[...] = x_sc[...].astype(bf)

        def sub(s, carry):
            xs = xb_sc[pl.ds(s * SPB, SPB), :]
            srcf = lax.dot_general(ohs_sc[pl.ds(s * SPB, SPB), :], xs, _TDIMS,
                                   preferred_element_type=f32).astype(bf)
            msg = srcf * ea_sc[pl.ds(s * EPB, EPB), :]
            agg_sc[pl.ds(s * SPB, SPB), :] = jnp.dot(
                ohd_sc[pl.ds(s * SPB, SPB), :], msg,
                preferred_element_type=f32).astype(bf)
            return carry

        lax.fori_loop(0, NSUB, sub, 0, unroll=2)

        h = jnp.maximum(
            jnp.dot(agg_sc[...], cw1_ref[l].astype(bf),
                    preferred_element_type=f32)
            + cb1_ref[l:l + 1, :], 0.0)
        x_sc[...] = (x_sc[...]
                     + jnp.dot(h.astype(bf), cw2_ref[l].astype(bf),
                               preferred_element_type=f32)
                     + cb2_ref[l:l + 1, :])

    x3 = x_sc[...].reshape(BN, K, D)
    xnode = x3[:, 0, :]
    for k in range(1, K):
        xnode = jnp.maximum(xnode, x3[:, k, :])

    bt = batch_ref[...].reshape(1, BN)
    ohb = (lax.broadcasted_iota(i32, (NG, BN), 0) == bt).astype(f32)
    contrib = jnp.dot(ohb, xnode, preferred_element_type=f32)

    @pl.when(b == 0)
    def _():
        hg_sc[...] = contrib

    @pl.when(b > 0)
    def _():
        hg_sc[...] = hg_sc[...] + contrib

    @pl.when(b == NB - 1)
    def _():
        out_ref[...] = (jnp.dot(hg_sc[...], pw_ref[...], preferred_element_type=f32)
                        + pb_ref[...])


def _full(shape):
    return pl.BlockSpec(shape, lambda i: (0,) * len(shape))


_main_call = pl.pallas_call(
    _main_body,
    grid=(NB,),
    in_specs=[
        pl.BlockSpec((1, 1, BNK), lambda i: (i, 0, 0)),
        pl.BlockSpec((1, 1, BNK), lambda i: (i, 0, 0)),
        pl.BlockSpec((1, 1, BNK), lambda i: (i, 0, 0)),
        pl.BlockSpec((1, 1, BNL), lambda i: (i, 0, 0)),
        pl.BlockSpec((1, 1, BNL), lambda i: (i, 0, 0)),
        pl.BlockSpec((1, 1, BNL), lambda i: (i, 0, 0)),
        pl.BlockSpec((1, 1, BN), lambda i: (i, 0, 0)),
        _full((32, D)), _full((16, D)), _full((16, D)),
        _full((D, D)), _full((1, D)), _full((D, D)), _full((1, D)),
        _full((NLAYER, D, D)), _full((NLAYER, D)),
        _full((NLAYER, D, D)), _full((NLAYER, D)),
        _full((D, 1)), _full((1, 1)),
    ],
    out_specs=pl.BlockSpec((NG, 1), lambda i: (0, 0)),
    out_shape=jax.ShapeDtypeStruct((NG, 1), jnp.float32),
    scratch_shapes=[
        pltpu.VMEM((BNK, D), jnp.float32),
        pltpu.VMEM((BNK, D), jnp.bfloat16),
        pltpu.VMEM((BNL, D), jnp.bfloat16),
        pltpu.VMEM((BNK, D), jnp.bfloat16),
        pltpu.VMEM((NG, D), jnp.float32),
        pltpu.VMEM((NSUB * SPB, EPB), jnp.bfloat16),
        pltpu.VMEM((NSUB * SPB, EPB), jnp.bfloat16),
    ],
    compiler_params=pltpu.CompilerParams(
        dimension_semantics=("arbitrary",),
        vmem_limit_bytes=100 * 1024 * 1024,
        fuse_transposed_lhs_in_matmul=True,
    ),
)


def kernel(x, subg_nodes, local_src, local_dst, local_attr, X_val, batch,
           x_table, ea_table, tf_table, lin0_W, lin0_b, lin1_W, lin1_b,
           conv_W1, conv_b1, conv_W2, conv_b2, pred_W, pred_b):
    i32 = jnp.int32
    xi = x.astype(i32)
    xi_pad = jnp.concatenate([xi, jnp.zeros((NPADSC - N,), i32)]).reshape(NPADSC // 128, 128)
    idx = subg_nodes.astype(i32).reshape(N * K)
    idx = jnp.concatenate([idx, jnp.zeros((BPAD - N * K,), i32)])
    gx = _sc_gather()(xi_pad, idx)[:N * K].reshape(NB, 1, BNK)

    xrep = jnp.repeat(xi, K).reshape(NB, 1, BNK)
    roff = (jnp.arange(N, dtype=i32)[:, None] % BSUB) * K
    gsrc = (roff + local_src.astype(i32)).reshape(NB, 1, BNL)
    gdst = (roff + local_dst.astype(i32)).reshape(NB, 1, BNL)
    out = _main_call(
        xrep, gx,
        X_val.astype(i32).reshape(NB, 1, BNK),
        gsrc, gdst,
        local_attr.astype(i32).reshape(NB, 1, BNL),
        batch.astype(i32).reshape(NB, 1, BN),
        x_table, ea_table, tf_table,
        lin0_W, lin0_b.reshape(1, D), lin1_W, lin1_b.reshape(1, D),
        conv_W1, conv_b1, conv_W2, conv_b2,
        pred_W, pred_b.reshape(1, 1),
    )
    return out


# speedup vs baseline: 1.9597x; 1.2571x over previous
import functools

import jax
import jax.numpy as jnp
from jax import lax
from jax.experimental import pallas as pl
from jax.experimental.pallas import tpu as pltpu
from jax.experimental.pallas import tpu_sc as plsc

N = 10000
K = 16
L = 32
D = 128
NLAYER = 3
NG = 64

SC_NC = 2
SC_NS = 16
NW = SC_NC * SC_NS
NPADSC = 10240
VPW = 5120
BPAD = NW * VPW
NV16 = VPW // 16

BN = 1000
NB = N // BN
BSUB = 8
NSUB = BN // BSUB
SPB = BSUB * K
EPB = BSUB * L
BNK = BN * K
BNL = BN * L

_TDIMS = (((0,), (0,)), ((), ()))


def _sc_gather_body(x_hbm, idx_hbm, out_hbm, x_v, idx_v, out_v):
    wid = lax.axis_index("s") * SC_NC + lax.axis_index("c")
    pltpu.sync_copy(x_hbm, x_v)
    pltpu.sync_copy(idx_hbm.at[pl.ds(wid * VPW, VPW)], idx_v)

    def body(m, carry):
        iv = idx_v[pl.ds(m * 16, 16)]
        out_v[pl.ds(m * 16, 16)] = plsc.load_gather(
            x_v, [lax.shift_right_logical(iv, 7), lax.bitwise_and(iv, 127)])
        return carry

    lax.fori_loop(0, NV16, body, 0, unroll=8)
    pltpu.sync_copy(out_v, out_hbm.at[pl.ds(wid * VPW, VPW)])


@functools.cache
def _sc_gather():
    return pl.kernel(
        _sc_gather_body,
        out_type=jax.ShapeDtypeStruct((BPAD,), jnp.int32),
        mesh=plsc.VectorSubcoreMesh(core_axis_name="c", subcore_axis_name="s"),
        scratch_types=[
            pltpu.VMEM((NPADSC // 128, 128), jnp.int32),
            pltpu.VMEM((VPW,), jnp.int32),
            pltpu.VMEM((VPW,), jnp.int32),
        ],
        compiler_params=pltpu.CompilerParams(needs_layout_passes=False),
    )


def _main_body(xrep_ref, gx_ref, xval_ref, gsrc_ref, gdst_ref, attr_ref, batch_ref,
               xtab_ref, eatab_ref, tftab_ref, l0w_ref, l0b_ref, l1w_ref, l1b_ref,
               cw1_ref, cb1_ref, cw2_ref, cb2_ref, pw_ref, pb_ref,
               out_ref, x_sc, xb_sc, ea_sc, agg_sc, hg_sc, ohs_sc, ohd_sc):
    f32 = jnp.float32
    i32 = jnp.int32
    bf = jnp.bfloat16
    b = pl.program_id(0)

    t0tab = (jnp.dot(xtab_ref[...], l0w_ref[...], preferred_element_type=f32)
             + l0b_ref[...]).astype(bf)
    t1tab = (jnp.dot(xtab_ref[...], l1w_ref[...], preferred_element_type=f32)
             + l1b_ref[...]).astype(bf)

    oha = (lax.broadcasted_iota(i32, (16, BNL), 0)
           == attr_ref[...].reshape(1, BNL)).astype(bf)
    ea_sc[...] = lax.dot_general(oha, eatab_ref[...].astype(bf), _TDIMS,
                                 preferred_element_type=f32).astype(bf)

    oh0 = (lax.broadcasted_iota(i32, (32, BNK), 0)
           == xrep_ref[...].reshape(1, BNK)).astype(bf)
    oh1 = (lax.broadcasted_iota(i32, (32, BNK), 0)
           == gx_ref[...].reshape(1, BNK)).astype(bf)
    oht = (lax.broadcasted_iota(i32, (16, BNK), 0)
           == xval_ref[...].reshape(1, BNK)).astype(bf)
    t0b = lax.dot_general(oh0, t0tab, _TDIMS, preferred_element_type=f32)
    t1b = lax.dot_general(oh1, t1tab, _TDIMS, preferred_element_type=f32)
    xt = lax.dot_general(oht, tftab_ref[...].astype(bf), _TDIMS,
                         preferred_element_type=f32)
    x_sc[...] = t0b * t1b * xt

    def build(s, carry):
        gs = gsrc_ref[:, :, pl.ds(s * EPB, EPB)].reshape(1, EPB)
        gsc = jnp.transpose(gs)
        ohs_sc[pl.ds(s * EPB, EPB), :] = (
            gsc == lax.broadcasted_iota(i32, (EPB, SPB), 1)).astype(bf)
        gd = gdst_ref[:, :, pl.ds(s * EPB, EPB)].reshape(1, EPB)
        ohd_sc[pl.ds(s * SPB, SPB), :] = (
            lax.broadcasted_iota(i32, (SPB, EPB), 0) == gd).astype(bf)
        return carry

    lax.fori_loop(0, NSUB, build, 0, unroll=2)

    for l in range(NLAYER):
        xb_sc[...] = x_sc[...].astype(bf)

        def sub(s, carry):
            xs = xb_sc[pl.ds(s * SPB, SPB), :]
            srcf = jnp.dot(ohs_sc[pl.ds(s * EPB, EPB), :], xs,
                           preferred_element_type=f32).astype(bf)
            msg = srcf * ea_sc[pl.ds(s * EPB, EPB), :]
            agg_sc[pl.ds(s * SPB, SPB), :] = jnp.dot(
                ohd_sc[pl.ds(s * SPB, SPB), :], msg,
                preferred_element_type=f32).astype(bf)
            return carry

        lax.fori_loop(0, NSUB, sub, 0, unroll=4)

        h = jnp.maximum(
            jnp.dot(agg_sc[...], cw1_ref[l].astype(bf),
                    preferred_element_type=f32)
            + cb1_ref[l:l + 1, :], 0.0)
        x_sc[...] = (x_sc[...]
                     + jnp.dot(h.astype(bf), cw2_ref[l].astype(bf),
                               preferred_element_type=f32)
                     + cb2_ref[l:l + 1, :])

    x3 = x_sc[...].reshape(BN, K, D)
    xnode = x3[:, 0, :]
    for k in range(1, K):
        xnode = jnp.maximum(xnode, x3[:, k, :])

    bt = batch_ref[...].reshape(1, BN)
    ohb = (lax.broadcasted_iota(i32, (NG, BN), 0) == bt).astype(f32)
    contrib = jnp.dot(ohb, xnode, preferred_element_type=f32)

    @pl.when(b == 0)
    def _():
        hg_sc[...] = contrib

    @pl.when(b > 0)
    def _():
        hg_sc[...] = hg_sc[...] + contrib

    @pl.when(b == NB - 1)
    def _():
        out_ref[...] = (jnp.dot(hg_sc[...], pw_ref[...], preferred_element_type=f32)
                        + pb_ref[...])


def _full(shape):
    return pl.BlockSpec(shape, lambda i: (0,) * len(shape))


_main_call = pl.pallas_call(
    _main_body,
    grid=(NB,),
    in_specs=[
        pl.BlockSpec((1, 1, BNK), lambda i: (i, 0, 0)),
        pl.BlockSpec((1, 1, BNK), lambda i: (i, 0, 0)),
        pl.BlockSpec((1, 1, BNK), lambda i: (i, 0, 0)),
        pl.BlockSpec((1, 1, BNL), lambda i: (i, 0, 0)),
        pl.BlockSpec((1, 1, BNL), lambda i: (i, 0, 0)),
        pl.BlockSpec((1, 1, BNL), lambda i: (i, 0, 0)),
        pl.BlockSpec((1, 1, BN), lambda i: (i, 0, 0)),
        _full((32, D)), _full((16, D)), _full((16, D)),
        _full((D, D)), _full((1, D)), _full((D, D)), _full((1, D)),
        _full((NLAYER, D, D)), _full((NLAYER, D)),
        _full((NLAYER, D, D)), _full((NLAYER, D)),
        _full((D, 1)), _full((1, 1)),
    ],
    out_specs=pl.BlockSpec((NG, 1), lambda i: (0, 0)),
    out_shape=jax.ShapeDtypeStruct((NG, 1), jnp.float32),
    scratch_shapes=[
        pltpu.VMEM((BNK, D), jnp.float32),
        pltpu.VMEM((BNK, D), jnp.bfloat16),
        pltpu.VMEM((BNL, D), jnp.bfloat16),
        pltpu.VMEM((BNK, D), jnp.bfloat16),
        pltpu.VMEM((NG, D), jnp.float32),
        pltpu.VMEM((NSUB * EPB, SPB), jnp.bfloat16),
        pltpu.VMEM((NSUB * SPB, EPB), jnp.bfloat16),
    ],
    compiler_params=pltpu.CompilerParams(
        dimension_semantics=("arbitrary",),
        vmem_limit_bytes=100 * 1024 * 1024,
        fuse_transposed_lhs_in_matmul=True,
    ),
)


def kernel(x, subg_nodes, local_src, local_dst, local_attr, X_val, batch,
           x_table, ea_table, tf_table, lin0_W, lin0_b, lin1_W, lin1_b,
           conv_W1, conv_b1, conv_W2, conv_b2, pred_W, pred_b):
    i32 = jnp.int32
    xi = x.astype(i32)
    xi_pad = jnp.concatenate([xi, jnp.zeros((NPADSC - N,), i32)]).reshape(NPADSC // 128, 128)
    idx = subg_nodes.astype(i32).reshape(N * K)
    idx = jnp.concatenate([idx, jnp.zeros((BPAD - N * K,), i32)])
    gx = _sc_gather()(xi_pad, idx)[:N * K].reshape(NB, 1, BNK)

    xrep = jnp.repeat(xi, K).reshape(NB, 1, BNK)
    roff = (jnp.arange(N, dtype=i32)[:, None] % BSUB) * K
    gsrc = (roff + local_src.astype(i32)).reshape(NB, 1, BNL)
    gdst = (roff + local_dst.astype(i32)).reshape(NB, 1, BNL)
    out = _main_call(
        xrep, gx,
        X_val.astype(i32).reshape(NB, 1, BNK),
        gsrc, gdst,
        local_attr.astype(i32).reshape(NB, 1, BNL),
        batch.astype(i32).reshape(NB, 1, BN),
        x_table, ea_table, tf_table,
        lin0_W, lin0_b.reshape(1, D), lin1_W, lin1_b.reshape(1, D),
        conv_W1, conv_b1, conv_W2, conv_b2,
        pred_W, pred_b.reshape(1, 1),
    )
    return out


# speedup vs baseline: 2.0693x; 1.0559x over previous
import functools

import jax
import jax.numpy as jnp
from jax import lax
from jax.experimental import pallas as pl
from jax.experimental.pallas import tpu as pltpu
from jax.experimental.pallas import tpu_sc as plsc

N = 10000
K = 16
L = 32
D = 128
NLAYER = 3
NG = 64

SC_NC = 2
SC_NS = 16
NW = SC_NC * SC_NS
NPADSC = 10240
VPW = 5120
BPAD = NW * VPW
NV16 = VPW // 16

BN = 1000
NB = N // BN
BSUB = 8
NSUB = BN // BSUB
SPB = BSUB * K
EPB = BSUB * L
BNK = BN * K
BNL = BN * L

_TDIMS = (((0,), (0,)), ((), ()))


def _sc_gather_body(x_hbm, idx_hbm, out_hbm, x_v, idx_v, out_v):
    wid = lax.axis_index("s") * SC_NC + lax.axis_index("c")
    pltpu.sync_copy(x_hbm, x_v)
    pltpu.sync_copy(idx_hbm.at[pl.ds(wid * VPW, VPW)], idx_v)

    def body(m, carry):
        iv = idx_v[pl.ds(m * 16, 16)]
        out_v[pl.ds(m * 16, 16)] = plsc.load_gather(
            x_v, [lax.shift_right_logical(iv, 7), lax.bitwise_and(iv, 127)])
        return carry

    lax.fori_loop(0, NV16, body, 0, unroll=8)
    pltpu.sync_copy(out_v, out_hbm.at[pl.ds(wid * VPW, VPW)])


@functools.cache
def _sc_gather():
    return pl.kernel(
        _sc_gather_body,
        out_type=jax.ShapeDtypeStruct((BPAD,), jnp.int32),
        mesh=plsc.VectorSubcoreMesh(core_axis_name="c", subcore_axis_name="s"),
        scratch_types=[
            pltpu.VMEM((NPADSC // 128, 128), jnp.int32),
            pltpu.VMEM((VPW,), jnp.int32),
            pltpu.VMEM((VPW,), jnp.int32),
        ],
        compiler_params=pltpu.CompilerParams(needs_layout_passes=False),
    )


def _main_body(xrep_ref, gx_ref, xval_ref, gsrc_ref, gdst_ref, attr_ref, batch_ref,
               xtab_ref, eatab_ref, tftab_ref, l0w_ref, l0b_ref, l1w_ref, l1b_ref,
               cw1_ref, cb1_ref, cw2_ref, cb2_ref, pw_ref, pb_ref,
               out_ref, x_sc, xb_sc, ea_sc, agg_sc, hg_sc, ohs_sc, ohd_sc):
    f32 = jnp.float32
    i32 = jnp.int32
    bf = jnp.bfloat16
    b = pl.program_id(0)

    t0tab = (jnp.dot(xtab_ref[...], l0w_ref[...], preferred_element_type=f32)
             + l0b_ref[...]).astype(bf)
    t1tab = (jnp.dot(xtab_ref[...], l1w_ref[...], preferred_element_type=f32)
             + l1b_ref[...]).astype(bf)

    oha = (lax.broadcasted_iota(i32, (16, BNL), 0)
           == attr_ref[...].reshape(1, BNL)).astype(bf)
    ea_sc[...] = lax.dot_general(oha, eatab_ref[...].astype(bf), _TDIMS,
                                 preferred_element_type=f32).astype(bf)

    oh0 = (lax.broadcasted_iota(i32, (32, BNK), 0)
           == xrep_ref[...].reshape(1, BNK)).astype(bf)
    oh1 = (lax.broadcasted_iota(i32, (32, BNK), 0)
           == gx_ref[...].reshape(1, BNK)).astype(bf)
    oht = (lax.broadcasted_iota(i32, (16, BNK), 0)
           == xval_ref[...].reshape(1, BNK)).astype(bf)
    t0b = lax.dot_general(oh0, t0tab, _TDIMS, preferred_element_type=f32)
    t1b = lax.dot_general(oh1, t1tab, _TDIMS, preferred_element_type=f32)
    xt = lax.dot_general(oht, tftab_ref[...].astype(bf), _TDIMS,
                         preferred_element_type=f32)
    x_sc[...] = t0b * t1b * xt

    def build(s, carry):
        gsc = jnp.transpose(gsrc_ref[:, pl.ds(s, 1), :].reshape(1, EPB))
        ohs_sc[pl.ds(s * EPB, EPB), :] = (
            gsc == lax.broadcasted_iota(i32, (EPB, SPB), 1)).astype(bf)
        gd = gdst_ref[:, pl.ds(s, 1), :].reshape(1, EPB)
        ohd_sc[pl.ds(s * SPB, SPB), :] = (
            lax.broadcasted_iota(i32, (SPB, EPB), 0) == gd).astype(bf)
        return carry

    lax.fori_loop(0, NSUB, build, 0, unroll=4)

    for l in range(NLAYER):
        xb_sc[...] = x_sc[...].astype(bf)

        def sub(s, carry):
            xs = xb_sc[pl.ds(s * SPB, SPB), :]
            srcf = jnp.dot(ohs_sc[pl.ds(s * EPB, EPB), :], xs,
                           preferred_element_type=f32).astype(bf)
            msg = srcf * ea_sc[pl.ds(s * EPB, EPB), :]
            agg_sc[pl.ds(s * SPB, SPB), :] = jnp.dot(
                ohd_sc[pl.ds(s * SPB, SPB), :], msg,
                preferred_element_type=f32).astype(bf)
            return carry

        lax.fori_loop(0, NSUB, sub, 0, unroll=4)

        h = jnp.maximum(
            jnp.dot(agg_sc[...], cw1_ref[l].astype(bf),
                    preferred_element_type=f32)
            + cb1_ref[l:l + 1, :], 0.0)
        x_sc[...] = (x_sc[...]
                     + jnp.dot(h.astype(bf), cw2_ref[l].astype(bf),
                               preferred_element_type=f32)
                     + cb2_ref[l:l + 1, :])

    x3 = x_sc[...].reshape(BN, K, D)
    xnode = x3[:, 0, :]
    for k in range(1, K):
        xnode = jnp.maximum(xnode, x3[:, k, :])

    bt = batch_ref[...].reshape(1, BN)
    ohb = (lax.broadcasted_iota(i32, (NG, BN), 0) == bt).astype(f32)
    contrib = jnp.dot(ohb, xnode, preferred_element_type=f32)

    @pl.when(b == 0)
    def _():
        hg_sc[...] = contrib

    @pl.when(b > 0)
    def _():
        hg_sc[...] = hg_sc[...] + contrib

    @pl.when(b == NB - 1)
    def _():
        out_ref[...] = (jnp.dot(hg_sc[...], pw_ref[...], preferred_element_type=f32)
                        + pb_ref[...])


def _full(shape):
    return pl.BlockSpec(shape, lambda i: (0,) * len(shape))


_main_call = pl.pallas_call(
    _main_body,
    grid=(NB,),
    in_specs=[
        pl.BlockSpec((1, 1, BNK), lambda i: (i, 0, 0)),
        pl.BlockSpec((1, 1, BNK), lambda i: (i, 0, 0)),
        pl.BlockSpec((1, 1, BNK), lambda i: (i, 0, 0)),
        pl.BlockSpec((1, NSUB, EPB), lambda i: (i, 0, 0)),
        pl.BlockSpec((1, NSUB, EPB), lambda i: (i, 0, 0)),
        pl.BlockSpec((1, 1, BNL), lambda i: (i, 0, 0)),
        pl.BlockSpec((1, 1, BN), lambda i: (i, 0, 0)),
        _full((32, D)), _full((16, D)), _full((16, D)),
        _full((D, D)), _full((1, D)), _full((D, D)), _full((1, D)),
        _full((NLAYER, D, D)), _full((NLAYER, D)),
        _full((NLAYER, D, D)), _full((NLAYER, D)),
        _full((D, 1)), _full((1, 1)),
    ],
    out_specs=pl.BlockSpec((NG, 1), lambda i: (0, 0)),
    out_shape=jax.ShapeDtypeStruct((NG, 1), jnp.float32),
    scratch_shapes=[
        pltpu.VMEM((BNK, D), jnp.float32),
        pltpu.VMEM((BNK, D), jnp.bfloat16),
        pltpu.VMEM((BNL, D), jnp.bfloat16),
        pltpu.VMEM((BNK, D), jnp.bfloat16),
        pltpu.VMEM((NG, D), jnp.float32),
        pltpu.VMEM((NSUB * EPB, SPB), jnp.bfloat16),
        pltpu.VMEM((NSUB * SPB, EPB), jnp.bfloat16),
    ],
    compiler_params=pltpu.CompilerParams(
        dimension_semantics=("arbitrary",),
        vmem_limit_bytes=100 * 1024 * 1024,
        fuse_transposed_lhs_in_matmul=True,
    ),
)


def kernel(x, subg_nodes, local_src, local_dst, local_attr, X_val, batch,
           x_table, ea_table, tf_table, lin0_W, lin0_b, lin1_W, lin1_b,
           conv_W1, conv_b1, conv_W2, conv_b2, pred_W, pred_b):
    i32 = jnp.int32
    xi = x.astype(i32)
    xi_pad = jnp.concatenate([xi, jnp.zeros((NPADSC - N,), i32)]).reshape(NPADSC // 128, 128)
    idx = subg_nodes.astype(i32).reshape(N * K)
    idx = jnp.concatenate([idx, jnp.zeros((BPAD - N * K,), i32)])
    gx = _sc_gather()(xi_pad, idx)[:N * K].reshape(NB, 1, BNK)

    xrep = jnp.repeat(xi, K).reshape(NB, 1, BNK)
    roff = (jnp.arange(N, dtype=i32)[:, None] % BSUB) * K
    gsrc = (roff + local_src.astype(i32)).reshape(NB, NSUB, EPB)
    gdst = (roff + local_dst.astype(i32)).reshape(NB, NSUB, EPB)
    out = _main_call(
        xrep, gx,
        X_val.astype(i32).reshape(NB, 1, BNK),
        gsrc, gdst,
        local_attr.astype(i32).reshape(NB, 1, BNL),
        batch.astype(i32).reshape(NB, 1, BN),
        x_table, ea_table, tf_table,
        lin0_W, lin0_b.reshape(1, D), lin1_W, lin1_b.reshape(1, D),
        conv_W1, conv_b1, conv_W2, conv_b2,
        pred_W, pred_b.reshape(1, 1),
    )
    return out


# speedup vs baseline: 2.5066x; 1.2113x over previous
import functools

import jax
import jax.numpy as jnp
from jax import lax
from jax.experimental import pallas as pl
from jax.experimental.pallas import tpu as pltpu
from jax.experimental.pallas import tpu_sc as plsc

N = 10000
K = 16
L = 32
D = 128
NLAYER = 3
NG = 64

SC_NC = 2
SC_NS = 16
NW = SC_NC * SC_NS
NPADSC = 10240
VPW = 5120
BPAD = NW * VPW
NV16 = VPW // 16

BN = 1000
NB = N // BN
BSUB = 8
NSUB = BN // BSUB
SPB = BSUB * K
EPB = BSUB * L
BNK = BN * K
BNL = BN * L

_TDIMS = (((0,), (0,)), ((), ()))


def _sc_gather_body(x_hbm, idx_hbm, out_hbm, x_v, idx_v, out_v):
    wid = lax.axis_index("s") * SC_NC + lax.axis_index("c")
    pltpu.sync_copy(x_hbm, x_v)
    pltpu.sync_copy(idx_hbm.at[pl.ds(wid * VPW, VPW)], idx_v)

    def body(m, carry):
        iv = idx_v[pl.ds(m * 16, 16)]
        out_v[pl.ds(m * 16, 16)] = plsc.load_gather(
            x_v, [lax.shift_right_logical(iv, 7), lax.bitwise_and(iv, 127)])
        return carry

    lax.fori_loop(0, NV16, body, 0, unroll=8)
    pltpu.sync_copy(out_v, out_hbm.at[pl.ds(wid * VPW, VPW)])


@functools.cache
def _sc_gather():
    return pl.kernel(
        _sc_gather_body,
        out_type=jax.ShapeDtypeStruct((BPAD,), jnp.int32),
        mesh=plsc.VectorSubcoreMesh(core_axis_name="c", subcore_axis_name="s"),
        scratch_types=[
            pltpu.VMEM((NPADSC // 128, 128), jnp.int32),
            pltpu.VMEM((VPW,), jnp.int32),
            pltpu.VMEM((VPW,), jnp.int32),
        ],
        compiler_params=pltpu.CompilerParams(needs_layout_passes=False),
    )


def _main_body(xrep_ref, gx_ref, xval_ref, gsrc_ref, gdst_ref, attr_ref, batch_ref,
               xtab_ref, eatab_ref, tftab_ref, l0w_ref, l0b_ref, l1w_ref, l1b_ref,
               cw1_ref, cb1_ref, cw2_ref, cb2_ref, pw_ref, pb_ref,
               out_ref, x_sc, xb_sc, ea_sc, agg_sc, hg_sc, ohs_sc, ohd_sc):
    f32 = jnp.float32
    i32 = jnp.int32
    bf = jnp.bfloat16
    b = pl.program_id(0)

    t0tab = (jnp.dot(xtab_ref[...], l0w_ref[...], preferred_element_type=f32)
             + l0b_ref[...]).astype(bf)
    t1tab = (jnp.dot(xtab_ref[...], l1w_ref[...], preferred_element_type=f32)
             + l1b_ref[...]).astype(bf)

    oha = (lax.broadcasted_iota(i32, (16, BNL), 0)
           == attr_ref[...].reshape(1, BNL)).astype(bf)
    ea_sc[...] = lax.dot_general(oha, eatab_ref[...].astype(bf), _TDIMS,
                                 preferred_element_type=f32).astype(bf)

    oh0 = (lax.broadcasted_iota(i32, (32, BNK), 0)
           == xrep_ref[...].reshape(1, BNK)).astype(bf)
    oh1 = (lax.broadcasted_iota(i32, (32, BNK), 0)
           == gx_ref[...].reshape(1, BNK)).astype(bf)
    oht = (lax.broadcasted_iota(i32, (16, BNK), 0)
           == xval_ref[...].reshape(1, BNK)).astype(bf)
    t0b = lax.dot_general(oh0, t0tab, _TDIMS, preferred_element_type=f32)
    t1b = lax.dot_general(oh1, t1tab, _TDIMS, preferred_element_type=f32)
    xt = lax.dot_general(oht, tftab_ref[...].astype(bf), _TDIMS,
                         preferred_element_type=f32)
    x_sc[...] = t0b * t1b * xt

    def build(s, carry):
        gsc = jnp.transpose(gsrc_ref[:, pl.ds(s, 1), :].reshape(1, EPB))
        ohs_sc[pl.ds(s * EPB, EPB), :] = (
            gsc == lax.broadcasted_iota(i32, (EPB, SPB), 1)).astype(bf)
        gd = gdst_ref[:, pl.ds(s, 1), :].reshape(1, EPB)
        ohd_sc[pl.ds(s * SPB, SPB), :] = (
            lax.broadcasted_iota(i32, (SPB, EPB), 0) == gd).astype(bf)
        return carry

    lax.fori_loop(0, NSUB, build, 0, unroll=4)

    for l in range(NLAYER):
        xb_sc[...] = x_sc[...].astype(bf)

        def sub(s, carry):
            xs = xb_sc[pl.ds(s * SPB, SPB), :]
            srcf = jnp.dot(ohs_sc[pl.ds(s * EPB, EPB), :], xs,
                           preferred_element_type=f32).astype(bf)
            msg = srcf * ea_sc[pl.ds(s * EPB, EPB), :]
            agg_sc[pl.ds(s * SPB, SPB), :] = jnp.dot(
                ohd_sc[pl.ds(s * SPB, SPB), :], msg,
                preferred_element_type=f32).astype(bf)
            return carry

        lax.fori_loop(0, NSUB, sub, 0, unroll=8)

        h = jnp.maximum(
            jnp.dot(agg_sc[...], cw1_ref[l].astype(bf),
                    preferred_element_type=f32)
            + cb1_ref[l:l + 1, :], 0.0)
        x_sc[...] = (x_sc[...]
                     + jnp.dot(h.astype(bf), cw2_ref[l].astype(bf),
                               preferred_element_type=f32)
                     + cb2_ref[l:l + 1, :])

    x3 = x_sc[...].reshape(BN, K, D)
    xnode = x3[:, 0, :]
    for k in range(1, K):
        xnode = jnp.maximum(xnode, x3[:, k, :])

    bt = batch_ref[...].reshape(1, BN)
    ohb = (lax.broadcasted_iota(i32, (NG, BN), 0) == bt).astype(f32)
    contrib = jnp.dot(ohb, xnode, preferred_element_type=f32)

    @pl.when(b == 0)
    def _():
        hg_sc[...] = contrib

    @pl.when(b > 0)
    def _():
        hg_sc[...] = hg_sc[...] + contrib

    @pl.when(b == NB - 1)
    def _():
        out_ref[...] = (jnp.dot(hg_sc[...], pw_ref[...], preferred_element_type=f32)
                        + pb_ref[...])


def _full(shape):
    return pl.BlockSpec(shape, lambda i: (0,) * len(shape))


_main_call = pl.pallas_call(
    _main_body,
    grid=(NB,),
    in_specs=[
        pl.BlockSpec((1, 1, BNK), lambda i: (i, 0, 0)),
        pl.BlockSpec((1, 1, BNK), lambda i: (i, 0, 0)),
        pl.BlockSpec((1, 1, BNK), lambda i: (i, 0, 0)),
        pl.BlockSpec((1, NSUB, EPB), lambda i: (i, 0, 0)),
        pl.BlockSpec((1, NSUB, EPB), lambda i: (i, 0, 0)),
        pl.BlockSpec((1, 1, BNL), lambda i: (i, 0, 0)),
        pl.BlockSpec((1, 1, BN), lambda i: (i, 0, 0)),
        _full((32, D)), _full((16, D)), _full((16, D)),
        _full((D, D)), _full((1, D)), _full((D, D)), _full((1, D)),
        _full((NLAYER, D, D)), _full((NLAYER, D)),
        _full((NLAYER, D, D)), _full((NLAYER, D)),
        _full((D, 1)), _full((1, 1)),
    ],
    out_specs=pl.BlockSpec((NG, 1), lambda i: (0, 0)),
    out_shape=jax.ShapeDtypeStruct((NG, 1), jnp.float32),
    scratch_shapes=[
        pltpu.VMEM((BNK, D), jnp.float32),
        pltpu.VMEM((BNK, D), jnp.bfloat16),
        pltpu.VMEM((BNL, D), jnp.bfloat16),
        pltpu.VMEM((BNK, D), jnp.bfloat16),
        pltpu.VMEM((NG, D), jnp.float32),
        pltpu.VMEM((NSUB * EPB, SPB), jnp.bfloat16),
        pltpu.VMEM((NSUB * SPB, EPB), jnp.bfloat16),
    ],
    compiler_params=pltpu.CompilerParams(
        dimension_semantics=("arbitrary",),
        vmem_limit_bytes=100 * 1024 * 1024,
        fuse_transposed_lhs_in_matmul=True,
    ),
)


def kernel(x, subg_nodes, local_src, local_dst, local_attr, X_val, batch,
           x_table, ea_table, tf_table, lin0_W, lin0_b, lin1_W, lin1_b,
           conv_W1, conv_b1, conv_W2, conv_b2, pred_W, pred_b):
    i32 = jnp.int32
    xi = x.astype(i32)
    xi_pad = jnp.concatenate([xi, jnp.zeros((NPADSC - N,), i32)]).reshape(NPADSC // 128, 128)
    idx = subg_nodes.astype(i32).reshape(N * K)
    idx = jnp.concatenate([idx, jnp.zeros((BPAD - N * K,), i32)])
    gx = _sc_gather()(xi_pad, idx)[:N * K].reshape(NB, 1, BNK)

    xrep = jnp.repeat(xi, K).reshape(NB, 1, BNK)
    roff = (jnp.arange(N, dtype=i32)[:, None] % BSUB) * K
    gsrc = (roff + local_src.astype(i32)).reshape(NB, NSUB, EPB)
    gdst = (roff + local_dst.astype(i32)).reshape(NB, NSUB, EPB)
    out = _main_call(
        xrep, gx,
        X_val.astype(i32).reshape(NB, 1, BNK),
        gsrc, gdst,
        local_attr.astype(i32).reshape(NB, 1, BNL),
        batch.astype(i32).reshape(NB, 1, BN),
        x_table, ea_table, tf_table,
        lin0_W, lin0_b.reshape(1, D), lin1_W, lin1_b.reshape(1, D),
        conv_W1, conv_b1, conv_W2, conv_b2,
        pred_W, pred_b.reshape(1, 1),
    )
    return out


# speedup vs baseline: 2.9509x; 1.1772x over previous
import functools

import jax
import jax.numpy as jnp
from jax import lax
from jax.experimental import pallas as pl
from jax.experimental.pallas import tpu as pltpu
from jax.experimental.pallas import tpu_sc as plsc

N = 10000
K = 16
L = 32
D = 128
NLAYER = 3
NG = 64

SC_NC = 2
SC_NS = 16
NW = SC_NC * SC_NS
NPADSC = 10240
VPW = 5120
BPAD = NW * VPW
NV16 = VPW // 16

BN = 1000
NB = N // BN
BSUB = 8
NSUB = BN // BSUB
SPB = BSUB * K
EPB = BSUB * L
BNK = BN * K
BNL = BN * L

_TDIMS = (((0,), (0,)), ((), ()))


def _sc_gather_body(x_hbm, idx_hbm, out_hbm, x_v, idx_v, out_v):
    wid = lax.axis_index("s") * SC_NC + lax.axis_index("c")
    pltpu.sync_copy(x_hbm, x_v)
    pltpu.sync_copy(idx_hbm.at[pl.ds(wid * VPW, VPW)], idx_v)

    def body(m, carry):
        iv = idx_v[pl.ds(m * 16, 16)]
        out_v[pl.ds(m * 16, 16)] = plsc.load_gather(
            x_v, [lax.shift_right_logical(iv, 7), lax.bitwise_and(iv, 127)])
        return carry

    lax.fori_loop(0, NV16, body, 0, unroll=8)
    pltpu.sync_copy(out_v, out_hbm.at[pl.ds(wid * VPW, VPW)])


@functools.cache
def _sc_gather():
    return pl.kernel(
        _sc_gather_body,
        out_type=jax.ShapeDtypeStruct((BPAD,), jnp.int32),
        mesh=plsc.VectorSubcoreMesh(core_axis_name="c", subcore_axis_name="s"),
        scratch_types=[
            pltpu.VMEM((NPADSC // 128, 128), jnp.int32),
            pltpu.VMEM((VPW,), jnp.int32),
            pltpu.VMEM((VPW,), jnp.int32),
        ],
        compiler_params=pltpu.CompilerParams(needs_layout_passes=False),
    )


def _main_body(xrep_ref, gx_ref, xval_ref, gsrc_ref, gdst_ref, attr_ref, batch_ref,
               xtab_ref, eatab_ref, tftab_ref, l0w_ref, l0b_ref, l1w_ref, l1b_ref,
               cw1_ref, cb1_ref, cw2_ref, cb2_ref, pw_ref, pb_ref,
               out_ref, x_sc, xb_sc, ea_sc, agg_sc, hg_sc, ohs_sc, ohd_sc):
    f32 = jnp.float32
    i32 = jnp.int32
    bf = jnp.bfloat16
    b = pl.program_id(0)

    t0tab = (jnp.dot(xtab_ref[...], l0w_ref[...], preferred_element_type=f32)
             + l0b_ref[...]).astype(bf)
    t1tab = (jnp.dot(xtab_ref[...], l1w_ref[...], preferred_element_type=f32)
             + l1b_ref[...]).astype(bf)

    oha = (lax.broadcasted_iota(i32, (16, BNL), 0)
           == attr_ref[...].reshape(1, BNL)).astype(bf)
    ea_sc[...] = lax.dot_general(oha, eatab_ref[...].astype(bf), _TDIMS,
                                 preferred_element_type=f32).astype(bf)

    oh0 = (lax.broadcasted_iota(i32, (32, BNK), 0)
           == xrep_ref[...].reshape(1, BNK)).astype(bf)
    oh1 = (lax.broadcasted_iota(i32, (32, BNK), 0)
           == gx_ref[...].reshape(1, BNK)).astype(bf)
    oht = (lax.broadcasted_iota(i32, (16, BNK), 0)
           == xval_ref[...].reshape(1, BNK)).astype(bf)
    t0b = lax.dot_general(oh0, t0tab, _TDIMS, preferred_element_type=f32)
    t1b = lax.dot_general(oh1, t1tab, _TDIMS, preferred_element_type=f32)
    xt = lax.dot_general(oht, tftab_ref[...].astype(bf), _TDIMS,
                         preferred_element_type=f32)
    x_sc[...] = t0b * t1b * xt

    def build(s, carry):
        gsc = jnp.transpose(gsrc_ref[:, pl.ds(s, 1), :].reshape(1, EPB))
        ohs_sc[pl.ds(s * EPB, EPB), :] = (
            gsc == lax.broadcasted_iota(i32, (EPB, SPB), 1)).astype(bf)
        gd = gdst_ref[:, pl.ds(s, 1), :].reshape(1, EPB)
        ohd_sc[pl.ds(s * SPB, SPB), :] = (
            lax.broadcasted_iota(i32, (SPB, EPB), 0) == gd).astype(bf)
        return carry

    lax.fori_loop(0, NSUB, build, 0, unroll=8)

    for l in range(NLAYER):
        xb_sc[...] = x_sc[...].astype(bf)

        def sub(s, carry):
            xs = xb_sc[pl.ds(s * SPB, SPB), :]
            srcf = jnp.dot(ohs_sc[pl.ds(s * EPB, EPB), :], xs,
                           preferred_element_type=f32).astype(bf)
            msg = srcf * ea_sc[pl.ds(s * EPB, EPB), :]
            agg_sc[pl.ds(s * SPB, SPB), :] = jnp.dot(
                ohd_sc[pl.ds(s * SPB, SPB), :], msg,
                preferred_element_type=f32).astype(bf)
            return carry

        lax.fori_loop(0, NSUB, sub, 0, unroll=16)

        h = jnp.maximum(
            jnp.dot(agg_sc[...], cw1_ref[l].astype(bf),
                    preferred_element_type=f32)
            + cb1_ref[l:l + 1, :], 0.0)
        x_sc[...] = (x_sc[...]
                     + jnp.dot(h.astype(bf), cw2_ref[l].astype(bf),
                               preferred_element_type=f32)
                     + cb2_ref[l:l + 1, :])

    x3 = x_sc[...].reshape(BN, K, D)
    xnode = x3[:, 0, :]
    for k in range(1, K):
        xnode = jnp.maximum(xnode, x3[:, k, :])

    bt = batch_ref[...].reshape(1, BN)
    ohb = (lax.broadcasted_iota(i32, (NG, BN), 0) == bt).astype(f32)
    contrib = jnp.dot(ohb, xnode, preferred_element_type=f32)

    @pl.when(b == 0)
    def _():
        hg_sc[...] = contrib

    @pl.when(b > 0)
    def _():
        hg_sc[...] = hg_sc[...] + contrib

    @pl.when(b == NB - 1)
    def _():
        out_ref[...] = (jnp.dot(hg_sc[...], pw_ref[...], preferred_element_type=f32)
                        + pb_ref[...])


def _full(shape):
    return pl.BlockSpec(shape, lambda i: (0,) * len(shape))


_main_call = pl.pallas_call(
    _main_body,
    grid=(NB,),
    in_specs=[
        pl.BlockSpec((1, 1, BNK), lambda i: (i, 0, 0)),
        pl.BlockSpec((1, 1, BNK), lambda i: (i, 0, 0)),
        pl.BlockSpec((1, 1, BNK), lambda i: (i, 0, 0)),
        pl.BlockSpec((1, NSUB, EPB), lambda i: (i, 0, 0)),
        pl.BlockSpec((1, NSUB, EPB), lambda i: (i, 0, 0)),
        pl.BlockSpec((1, 1, BNL), lambda i: (i, 0, 0)),
        pl.BlockSpec((1, 1, BN), lambda i: (i, 0, 0)),
        _full((32, D)), _full((16, D)), _full((16, D)),
        _full((D, D)), _full((1, D)), _full((D, D)), _full((1, D)),
        _full((NLAYER, D, D)), _full((NLAYER, D)),
        _full((NLAYER, D, D)), _full((NLAYER, D)),
        _full((D, 1)), _full((1, 1)),
    ],
    out_specs=pl.BlockSpec((NG, 1), lambda i: (0, 0)),
    out_shape=jax.ShapeDtypeStruct((NG, 1), jnp.float32),
    scratch_shapes=[
        pltpu.VMEM((BNK, D), jnp.float32),
        pltpu.VMEM((BNK, D), jnp.bfloat16),
        pltpu.VMEM((BNL, D), jnp.bfloat16),
        pltpu.VMEM((BNK, D), jnp.bfloat16),
        pltpu.VMEM((NG, D), jnp.float32),
        pltpu.VMEM((NSUB * EPB, SPB), jnp.bfloat16),
        pltpu.VMEM((NSUB * SPB, EPB), jnp.bfloat16),
    ],
    compiler_params=pltpu.CompilerParams(
        dimension_semantics=("arbitrary",),
        vmem_limit_bytes=100 * 1024 * 1024,
        fuse_transposed_lhs_in_matmul=True,
    ),
)


def kernel(x, subg_nodes, local_src, local_dst, local_attr, X_val, batch,
           x_table, ea_table, tf_table, lin0_W, lin0_b, lin1_W, lin1_b,
           conv_W1, conv_b1, conv_W2, conv_b2, pred_W, pred_b):
    i32 = jnp.int32
    xi = x.astype(i32)
    xi_pad = jnp.concatenate([xi, jnp.zeros((NPADSC - N,), i32)]).reshape(NPADSC // 128, 128)
    idx = subg_nodes.astype(i32).reshape(N * K)
    idx = jnp.concatenate([idx, jnp.zeros((BPAD - N * K,), i32)])
    gx = _sc_gather()(xi_pad, idx)[:N * K].reshape(NB, 1, BNK)

    xrep = jnp.repeat(xi, K).reshape(NB, 1, BNK)
    roff = (jnp.arange(N, dtype=i32)[:, None] % BSUB) * K
    gsrc = (roff + local_src.astype(i32)).reshape(NB, NSUB, EPB)
    gdst = (roff + local_dst.astype(i32)).reshape(NB, NSUB, EPB)
    out = _main_call(
        xrep, gx,
        X_val.astype(i32).reshape(NB, 1, BNK),
        gsrc, gdst,
        local_attr.astype(i32).reshape(NB, 1, BNL),
        batch.astype(i32).reshape(NB, 1, BN),
        x_table, ea_table, tf_table,
        lin0_W, lin0_b.reshape(1, D), lin1_W, lin1_b.reshape(1, D),
        conv_W1, conv_b1, conv_W2, conv_b2,
        pred_W, pred_b.reshape(1, 1),
    )
    return out


# speedup vs baseline: 3.1045x; 1.0521x over previous
import functools

import jax
import jax.numpy as jnp
from jax import lax
from jax.experimental import pallas as pl
from jax.experimental.pallas import tpu as pltpu
from jax.experimental.pallas import tpu_sc as plsc

N = 10000
K = 16
L = 32
D = 128
NLAYER = 3
NG = 64

SC_NC = 2
SC_NS = 16
NW = SC_NC * SC_NS
NPADSC = 10240
VPW = 5120
BPAD = NW * VPW
NV16 = VPW // 16

BN = 1000
NB = N // BN
BSUB = 8
NSUB = BN // BSUB
SPB = BSUB * K
EPB = BSUB * L
BNK = BN * K
BNL = BN * L

_TDIMS = (((0,), (0,)), ((), ()))


def _sc_gather_body(x_hbm, idx_hbm, out_hbm, x_v, idx_v, out_v):
    wid = lax.axis_index("s") * SC_NC + lax.axis_index("c")
    pltpu.sync_copy(x_hbm, x_v)
    pltpu.sync_copy(idx_hbm.at[pl.ds(wid * VPW, VPW)], idx_v)

    def body(m, carry):
        iv = idx_v[pl.ds(m * 16, 16)]
        out_v[pl.ds(m * 16, 16)] = plsc.load_gather(
            x_v, [lax.shift_right_logical(iv, 7), lax.bitwise_and(iv, 127)])
        return carry

    lax.fori_loop(0, NV16, body, 0, unroll=8)
    pltpu.sync_copy(out_v, out_hbm.at[pl.ds(wid * VPW, VPW)])


@functools.cache
def _sc_gather():
    return pl.kernel(
        _sc_gather_body,
        out_type=jax.ShapeDtypeStruct((BPAD,), jnp.int32),
        mesh=plsc.VectorSubcoreMesh(core_axis_name="c", subcore_axis_name="s"),
        scratch_types=[
            pltpu.VMEM((NPADSC // 128, 128), jnp.int32),
            pltpu.VMEM((VPW,), jnp.int32),
            pltpu.VMEM((VPW,), jnp.int32),
        ],
        compiler_params=pltpu.CompilerParams(needs_layout_passes=False),
    )


def _main_body(xrep_ref, gx_ref, xval_ref, gsrc_ref, gdst_ref, attr_ref, batch_ref,
               xtab_ref, eatab_ref, tftab_ref, l0w_ref, l0b_ref, l1w_ref, l1b_ref,
               cw1_ref, cb1_ref, cw2_ref, cb2_ref, pw_ref, pb_ref,
               out_ref, x_sc, xb_sc, ea_sc, agg_sc, hg_sc, ohs_sc, ohd_sc):
    f32 = jnp.float32
    i32 = jnp.int32
    bf = jnp.bfloat16
    b = pl.program_id(0)

    t0tab = (jnp.dot(xtab_ref[...], l0w_ref[...], preferred_element_type=f32)
             + l0b_ref[...]).astype(bf)
    t1tab = (jnp.dot(xtab_ref[...], l1w_ref[...], preferred_element_type=f32)
             + l1b_ref[...]).astype(bf)

    oha = (lax.broadcasted_iota(i32, (16, BNL), 0)
           == attr_ref[...].reshape(1, BNL)).astype(bf)
    ea_sc[...] = lax.dot_general(oha, eatab_ref[...].astype(bf), _TDIMS,
                                 preferred_element_type=f32).astype(bf)

    oh0 = (lax.broadcasted_iota(i32, (32, BNK), 0)
           == xrep_ref[...].reshape(1, BNK)).astype(bf)
    oh1 = (lax.broadcasted_iota(i32, (32, BNK), 0)
           == gx_ref[...].reshape(1, BNK)).astype(bf)
    oht = (lax.broadcasted_iota(i32, (16, BNK), 0)
           == xval_ref[...].reshape(1, BNK)).astype(bf)
    t0b = lax.dot_general(oh0, t0tab, _TDIMS, preferred_element_type=f32)
    t1b = lax.dot_general(oh1, t1tab, _TDIMS, preferred_element_type=f32)
    xt = lax.dot_general(oht, tftab_ref[...].astype(bf), _TDIMS,
                         preferred_element_type=f32)
    x_sc[...] = t0b * t1b * xt

    def build(s, carry):
        gsc = jnp.transpose(gsrc_ref[:, pl.ds(s, 1), :].reshape(1, EPB))
        ohs_sc[pl.ds(s * EPB, EPB), :] = (
            gsc == lax.broadcasted_iota(i32, (EPB, SPB), 1)).astype(bf)
        gd = gdst_ref[:, pl.ds(s, 1), :].reshape(1, EPB)
        ohd_sc[pl.ds(s * SPB, SPB), :] = (
            lax.broadcasted_iota(i32, (SPB, EPB), 0) == gd).astype(bf)
        return carry

    lax.fori_loop(0, NSUB, build, 0, unroll=25)

    for l in range(NLAYER):
        xb_sc[...] = x_sc[...].astype(bf)

        def sub(s, carry):
            xs = xb_sc[pl.ds(s * SPB, SPB), :]
            srcf = jnp.dot(ohs_sc[pl.ds(s * EPB, EPB), :], xs,
                           preferred_element_type=f32).astype(bf)
            msg = srcf * ea_sc[pl.ds(s * EPB, EPB), :]
            agg_sc[pl.ds(s * SPB, SPB), :] = jnp.dot(
                ohd_sc[pl.ds(s * SPB, SPB), :], msg,
                preferred_element_type=f32).astype(bf)
            return carry

        lax.fori_loop(0, NSUB, sub, 0, unroll=25)

        h = jnp.maximum(
            jnp.dot(agg_sc[...], cw1_ref[l].astype(bf),
                    preferred_element_type=f32)
            + cb1_ref[l:l + 1, :], 0.0)
        x_sc[...] = (x_sc[...]
                     + jnp.dot(h.astype(bf), cw2_ref[l].astype(bf),
                               preferred_element_type=f32)
                     + cb2_ref[l:l + 1, :])

    x3 = x_sc[...].reshape(BN, K, D)
    xnode = x3[:, 0, :]
    for k in range(1, K):
        xnode = jnp.maximum(xnode, x3[:, k, :])

    bt = batch_ref[...].reshape(1, BN)
    ohb = (lax.broadcasted_iota(i32, (NG, BN), 0) == bt).astype(f32)
    contrib = jnp.dot(ohb, xnode, preferred_element_type=f32)

    @pl.when(b == 0)
    def _():
        hg_sc[...] = contrib

    @pl.when(b > 0)
    def _():
        hg_sc[...] = hg_sc[...] + contrib

    @pl.when(b == NB - 1)
    def _():
        out_ref[...] = (jnp.dot(hg_sc[...], pw_ref[...], preferred_element_type=f32)
                        + pb_ref[...])


def _full(shape):
    return pl.BlockSpec(shape, lambda i: (0,) * len(shape))


_main_call = pl.pallas_call(
    _main_body,
    grid=(NB,),
    in_specs=[
        pl.BlockSpec((1, 1, BNK), lambda i: (i, 0, 0)),
        pl.BlockSpec((1, 1, BNK), lambda i: (i, 0, 0)),
        pl.BlockSpec((1, 1, BNK), lambda i: (i, 0, 0)),
        pl.BlockSpec((1, NSUB, EPB), lambda i: (i, 0, 0)),
        pl.BlockSpec((1, NSUB, EPB), lambda i: (i, 0, 0)),
        pl.BlockSpec((1, 1, BNL), lambda i: (i, 0, 0)),
        pl.BlockSpec((1, 1, BN), lambda i: (i, 0, 0)),
        _full((32, D)), _full((16, D)), _full((16, D)),
        _full((D, D)), _full((1, D)), _full((D, D)), _full((1, D)),
        _full((NLAYER, D, D)), _full((NLAYER, D)),
        _full((NLAYER, D, D)), _full((NLAYER, D)),
        _full((D, 1)), _full((1, 1)),
    ],
    out_specs=pl.BlockSpec((NG, 1), lambda i: (0, 0)),
    out_shape=jax.ShapeDtypeStruct((NG, 1), jnp.float32),
    scratch_shapes=[
        pltpu.VMEM((BNK, D), jnp.float32),
        pltpu.VMEM((BNK, D), jnp.bfloat16),
        pltpu.VMEM((BNL, D), jnp.bfloat16),
        pltpu.VMEM((BNK, D), jnp.bfloat16),
        pltpu.VMEM((NG, D), jnp.float32),
        pltpu.VMEM((NSUB * EPB, SPB), jnp.bfloat16),
        pltpu.VMEM((NSUB * SPB, EPB), jnp.bfloat16),
    ],
    compiler_params=pltpu.CompilerParams(
        dimension_semantics=("arbitrary",),
        vmem_limit_bytes=100 * 1024 * 1024,
        fuse_transposed_lhs_in_matmul=True,
    ),
)


def kernel(x, subg_nodes, local_src, local_dst, local_attr, X_val, batch,
           x_table, ea_table, tf_table, lin0_W, lin0_b, lin1_W, lin1_b,
           conv_W1, conv_b1, conv_W2, conv_b2, pred_W, pred_b):
    i32 = jnp.int32
    xi = x.astype(i32)
    xi_pad = jnp.concatenate([xi, jnp.zeros((NPADSC - N,), i32)]).reshape(NPADSC // 128, 128)
    idx = subg_nodes.astype(i32).reshape(N * K)
    idx = jnp.concatenate([idx, jnp.zeros((BPAD - N * K,), i32)])
    gx = _sc_gather()(xi_pad, idx)[:N * K].reshape(NB, 1, BNK)

    xrep = jnp.repeat(xi, K).reshape(NB, 1, BNK)
    roff = (jnp.arange(N, dtype=i32)[:, None] % BSUB) * K
    gsrc = (roff + local_src.astype(i32)).reshape(NB, NSUB, EPB)
    gdst = (roff + local_dst.astype(i32)).reshape(NB, NSUB, EPB)
    out = _main_call(
        xrep, gx,
        X_val.astype(i32).reshape(NB, 1, BNK),
        gsrc, gdst,
        local_attr.astype(i32).reshape(NB, 1, BNL),
        batch.astype(i32).reshape(NB, 1, BN),
        x_table, ea_table, tf_table,
        lin0_W, lin0_b.reshape(1, D), lin1_W, lin1_b.reshape(1, D),
        conv_W1, conv_b1, conv_W2, conv_b2,
        pred_W, pred_b.reshape(1, 1),
    )
    return out
